# Initial kernel scaffold; baseline (speedup 1.0000x reference)
import jax
import jax.numpy as jnp
from jax.experimental import pallas as pl


def kernel(sim0, sim1, sim2, sim3):
    raise NotImplementedError("write your pallas kernel here")


# speedup vs baseline: 1.4522x; 1.4522x over previous
import numpy as np
import jax
import jax.numpy as jnp
from jax.experimental import pallas as pl

_SCALES = (32, 48, 64, 96)
_STRIDE = 16
_H, _W = 96, 128
_HW = _H * _W
_KPAD = 128


def _build_weight_matrix():
    h = np.arange(_H)
    w = np.arange(_W)
    M = np.zeros((_KPAD, 4 * _HW), dtype=np.float64)
    off = 0
    for i, s in enumerate(_SCALES):
        ny = (_H - s) // _STRIDE + 1
        nx = (_W - s) // _STRIDE + 1
        y = np.arange(ny) * _STRIDE
        x = np.arange(nx) * _STRIDE
        Ry = ((h[None, :] >= y[:, None]) & (h[None, :] < y[:, None] + s)).astype(np.float64)
        Cx = ((w[None, :] >= x[:, None]) & (w[None, :] < x[:, None] + s)).astype(np.float64)
        count = Ry.sum(0)[:, None] * Cx.sum(0)[None, :]
        wmat = Ry[:, None, :, None] * Cx[None, :, None, :]
        wmat = wmat / (count[None, None] + 1e-30)
        M[off:off + ny * nx, i * _HW:(i + 1) * _HW] = wmat.reshape(ny * nx, _HW)
        off += ny * nx
    return M.astype(np.float32)


_M_NP = _build_weight_matrix()


def _recon_body(s_ref, m_ref, o_ref):
    o_ref[...] = jnp.dot(s_ref[...], m_ref[...], preferred_element_type=jnp.float32)


def kernel(sim0, sim1, sim2, sim3):
    B = sim0.shape[0]
    parts = [s.reshape(B, -1) for s in (sim0, sim1, sim2, sim3)]
    scat = jnp.concatenate(parts, axis=1)
    scat = jnp.pad(scat, ((0, 0), (0, _KPAD - scat.shape[1])))
    M = jnp.asarray(_M_NP)
    BB = 64
    out = pl.pallas_call(
        _recon_body,
        grid=(B // BB,),
        in_specs=[
            pl.BlockSpec((BB, _KPAD), lambda i: (i, 0)),
            pl.BlockSpec((_KPAD, 4 * _HW), lambda i: (0, 0)),
        ],
        out_specs=pl.BlockSpec((BB, 4 * _HW), lambda i: (i, 0)),
        out_shape=jax.ShapeDtypeStruct((B, 4 * _HW), jnp.float32),
    )(scat, M)
    return out.reshape(B, 4, _H, _W)


# speedup vs baseline: 1.4907x; 1.0266x over previous
import numpy as np
import jax
import jax.numpy as jnp
from jax.experimental import pallas as pl

_SCALES = (32, 48, 64, 96)
_STRIDE = 16
_H, _W = 96, 128
_HW = _H * _W
_KPAD = 128


def _build_weight_matrix():
    h = np.arange(_H)
    w = np.arange(_W)
    M = np.zeros((_KPAD, 4 * _HW), dtype=np.float64)
    off = 0
    for i, s in enumerate(_SCALES):
        ny = (_H - s) // _STRIDE + 1
        nx = (_W - s) // _STRIDE + 1
        y = np.arange(ny) * _STRIDE
        x = np.arange(nx) * _STRIDE
        Ry = ((h[None, :] >= y[:, None]) & (h[None, :] < y[:, None] + s)).astype(np.float64)
        Cx = ((w[None, :] >= x[:, None]) & (w[None, :] < x[:, None] + s)).astype(np.float64)
        count = Ry.sum(0)[:, None] * Cx.sum(0)[None, :]
        wmat = Ry[:, None, :, None] * Cx[None, :, None, :]
        wmat = wmat / (count[None, None] + 1e-30)
        M[off:off + ny * nx, i * _HW:(i + 1) * _HW] = wmat.reshape(ny * nx, _HW)
        off += ny * nx
    return M.astype(np.float32)


_M_NP = _build_weight_matrix()


def _recon_body(s_ref, m_ref, o_ref):
    o_ref[...] = jnp.dot(s_ref[...], m_ref[...], preferred_element_type=jnp.float32)


def kernel(sim0, sim1, sim2, sim3):
    B = sim0.shape[0]
    parts = [s.reshape(B, -1) for s in (sim0, sim1, sim2, sim3)]
    scat = jnp.concatenate(parts, axis=1)
    scat = jnp.pad(scat, ((0, 0), (0, _KPAD - scat.shape[1]))).astype(jnp.bfloat16)
    M = jnp.asarray(_M_NP, dtype=jnp.bfloat16)
    BB = 64
    out = pl.pallas_call(
        _recon_body,
        grid=(B // BB,),
        in_specs=[
            pl.BlockSpec((BB, _KPAD), lambda i: (i, 0)),
            pl.BlockSpec((_KPAD, 4 * _HW), lambda i: (0, 0)),
        ],
        out_specs=pl.BlockSpec((BB, 4 * _HW), lambda i: (i, 0)),
        out_shape=jax.ShapeDtypeStruct((B, 4 * _HW), jnp.float32),
    )(scat, M)
    return out.reshape(B, 4, _H, _W)


# speedup vs baseline: 4.4822x; 3.0067x over previous
import numpy as np
import jax
import jax.numpy as jnp
from jax.experimental import pallas as pl

_SCALES = (32, 48, 64, 96)
_STRIDE = 16
_H, _W = 96, 128
_NWIN = [( (_H - s) // _STRIDE + 1, (_W - s) // _STRIDE + 1) for s in _SCALES]
_NP_TOT = sum(ny * nx for ny, nx in _NWIN)
_KPAD = 80
_ROWS = 4 * _H


def _build_weights():
    h = np.arange(_H)
    w = np.arange(_W)
    rmask = np.zeros((_ROWS, _KPAD), dtype=np.float64)
    wc = np.zeros((_KPAD, _W), dtype=np.float64)
    off = 0
    for i, s in enumerate(_SCALES):
        ny, nx = _NWIN[i]
        y = np.arange(ny) * _STRIDE
        x = np.arange(nx) * _STRIDE
        Ry = ((h[None, :] >= y[:, None]) & (h[None, :] < y[:, None] + s)).astype(np.float64)
        Cx = ((w[None, :] >= x[:, None]) & (w[None, :] < x[:, None] + s)).astype(np.float64)
        county = Ry.sum(0)
        countx = Cx.sum(0)
        for yy in range(ny):
            for xx in range(nx):
                p = off + yy * nx + xx
                rmask[i * _H:(i + 1) * _H, p] = Ry[yy] / county
                wc[p, :] = Cx[xx] / countx
        off += ny * nx
    return rmask.astype(np.float32), wc.astype(np.float32)


_RMASK_NP, _WC_NP = _build_weights()


def _recon_body(s_ref, rm_ref, wc_ref, o_ref):
    bb = s_ref.shape[0]
    t = (s_ref[...][:, None, :] * rm_ref[...][None, :, :]).reshape(bb * _ROWS, _KPAD)
    r = jnp.dot(t, wc_ref[...], preferred_element_type=jnp.float32)
    o_ref[...] = r.reshape(bb, 4, _H, _W)


def kernel(sim0, sim1, sim2, sim3):
    B = sim0.shape[0]
    parts = [s.reshape(B, -1) for s in (sim0, sim1, sim2, sim3)]
    scat = jnp.concatenate(parts, axis=1)
    scat = jnp.pad(scat, ((0, 0), (0, _KPAD - scat.shape[1])))
    rmask = jnp.asarray(_RMASK_NP)
    wc = jnp.asarray(_WC_NP)
    BB = 16
    out = pl.pallas_call(
        _recon_body,
        grid=(B // BB,),
        in_specs=[
            pl.BlockSpec((BB, _KPAD), lambda i: (i, 0)),
            pl.BlockSpec((_ROWS, _KPAD), lambda i: (0, 0)),
            pl.BlockSpec((_KPAD, _W), lambda i: (0, 0)),
        ],
        out_specs=pl.BlockSpec((BB, 4, _H, _W), lambda i: (i, 0, 0, 0)),
        out_shape=jax.ShapeDtypeStruct((B, 4, _H, _W), jnp.float32),
    )(scat, rmask, wc)
    return out


# speedup vs baseline: 5.1516x; 1.1493x over previous
import numpy as np
import jax
import jax.numpy as jnp
from jax.experimental import pallas as pl

_SCALES = (32, 48, 64, 96)
_STRIDE = 16
_H, _W = 96, 128
_NWIN = [( (_H - s) // _STRIDE + 1, (_W - s) // _STRIDE + 1) for s in _SCALES]
_NP_TOT = sum(ny * nx for ny, nx in _NWIN)
_KPAD = 80
_ROWS = 4 * _H


def _build_weights():
    h = np.arange(_H)
    w = np.arange(_W)
    rmask = np.zeros((_ROWS, _KPAD), dtype=np.float64)
    wc = np.zeros((_KPAD, _W), dtype=np.float64)
    off = 0
    for i, s in enumerate(_SCALES):
        ny, nx = _NWIN[i]
        y = np.arange(ny) * _STRIDE
        x = np.arange(nx) * _STRIDE
        Ry = ((h[None, :] >= y[:, None]) & (h[None, :] < y[:, None] + s)).astype(np.float64)
        Cx = ((w[None, :] >= x[:, None]) & (w[None, :] < x[:, None] + s)).astype(np.float64)
        county = Ry.sum(0)
        countx = Cx.sum(0)
        for yy in range(ny):
            for xx in range(nx):
                p = off + yy * nx + xx
                rmask[i * _H:(i + 1) * _H, p] = Ry[yy] / county
                wc[p, :] = Cx[xx] / countx
        off += ny * nx
    return rmask.astype(np.float32), wc.astype(np.float32)


_RMASK_NP, _WC_NP = _build_weights()


def _recon_body(s_ref, rm_ref, wc_ref, o_ref):
    bb = s_ref.shape[0]
    t = (s_ref[...][:, None, :] * rm_ref[...][None, :, :]).reshape(bb * _ROWS, _KPAD)
    r = jnp.dot(t, wc_ref[...], preferred_element_type=jnp.float32)
    o_ref[...] = r.reshape(bb, 4, _H, _W)


def kernel(sim0, sim1, sim2, sim3):
    B = sim0.shape[0]
    parts = [s.reshape(B, -1) for s in (sim0, sim1, sim2, sim3)]
    scat = jnp.concatenate(parts, axis=1)
    scat = jnp.pad(scat, ((0, 0), (0, _KPAD - scat.shape[1])))
    rmask = jnp.asarray(_RMASK_NP)
    wc = jnp.asarray(_WC_NP)
    BB = 32
    out = pl.pallas_call(
        _recon_body,
        grid=(B // BB,),
        in_specs=[
            pl.BlockSpec((BB, _KPAD), lambda i: (i, 0)),
            pl.BlockSpec((_ROWS, _KPAD), lambda i: (0, 0)),
            pl.BlockSpec((_KPAD, _W), lambda i: (0, 0)),
        ],
        out_specs=pl.BlockSpec((BB, 4, _H, _W), lambda i: (i, 0, 0, 0)),
        out_shape=jax.ShapeDtypeStruct((B, 4, _H, _W), jnp.float32),
    )(scat, rmask, wc)
    return out
